# baseline (speedup 1.0000x reference)
import functools

import jax
import jax.numpy as jnp
from jax import lax
from jax.experimental import pallas as pl
from jax.experimental.pallas import tpu as pltpu
from jax.experimental.pallas import tpu_sc as plsc

B, C, H, W = 8, 3, 512, 512
G = 8
K = 64
NB = 256
PIX = K * K
MAXV = 640.0
SCALE = (NB - 1.0) / PIX
NIMG = B * C
NCORE, NSUB = 2, 16
IMG_PER_CORE = NIMG // NCORE
ITEMS_PER_SUB = IMG_PER_CORE * G // NSUB
LANE = 16
CHUNKS = W // LANE
LUT_WORDS = G * G * NB


def _body(img_hbm, out_hbm, lut_hbm, pix_v, outb_v, hist_v, lutb_v, ph_v,
          luti_v, p4_v, wx_v):
    core = lax.axis_index("c")
    sub = lax.axis_index("s")

    for ck in range(CHUNKS):
        xx = lax.iota(jnp.int32, LANE).astype(jnp.float32) + float(ck * LANE)
        sx = jnp.clip((xx + 0.5) * (1.0 / K) - 0.5, 0.0, G - 1.0)
        s0 = sx.astype(jnp.int32)
        wx_v[pl.ds(ck * LANE, LANE)] = sx - s0.astype(jnp.float32)

    zeros = jnp.zeros((LANE,), jnp.float32)
    ones = jnp.ones((LANE,), jnp.float32)

    @pl.loop(0, ITEMS_PER_SUB)
    def _phase1(it):
        item = sub * ITEMS_PER_SUB + it
        il = item // G
        band = item % G
        g = core * IMG_PER_CORE + il
        pltpu.sync_copy(img_hbm.at[g, pl.ds(band * K, K), :], pix_v)
        for i in range(G * NB // LANE):
            hist_v[pl.ds(i * LANE, LANE)] = zeros

        @plsc.parallel_loop(0, K, 1, unroll=2)
        def _rows(r):
            for ckx in range(CHUNKS):
                ck = (ckx % 8) * 4 + ckx // 8
                px = pix_v[r, pl.ds(ck * LANE, LANE)]
                pb = (px * NB).astype(jnp.int32)
                plsc.addupdate_scatter(
                    hist_v.at[pl.ds((ck // 4) * NB, NB)], [pb], ones)

        for t in range(G):
            acc = zeros
            for i in range(NB // LANE):
                h = jnp.minimum(hist_v[pl.ds(t * NB + i * LANE, LANE)], MAXV)
                acc = acc + h
            tot = jnp.sum(acc)
            clipped_i = PIX - tot.astype(jnp.int32)
            redist_i = lax.shift_right_logical(clipped_i, 8)
            redist = redist_i.astype(jnp.float32)
            residual = (clipped_i - redist_i * NB).astype(jnp.float32)
            carry = 0.0
            for i in range(NB // LANE):
                h = jnp.minimum(hist_v[pl.ds(t * NB + i * LANE, LANE)], MAXV)
                vr = lax.iota(jnp.int32, LANE).astype(jnp.float32) + float(i * LANE)
                h = h + redist + jnp.where(vr < residual, 1.0, 0.0)
                cs = plsc.cumsum(h) + carry
                carry = cs[LANE - 1]
                lut = jnp.clip(cs * SCALE, 0.0, NB - 1.0)
                lutb_v[pl.ds(t * NB + i * LANE, LANE)] = lut.astype(jnp.int32)
        for ckk in range(G * NB // LANE):
            tx = ckk // (NB // LANE)
            off = (ckk % (NB // LANE)) * LANE
            av = lutb_v[pl.ds(tx * NB + off, LANE)]
            bv = lutb_v[pl.ds(min(tx + 1, G - 1) * NB + off, LANE)]
            ph_v[pl.ds(ckk * LANE, LANE)] = jnp.bitwise_or(
                av, lax.shift_left(bv, 8))
        pltpu.sync_copy(ph_v, lut_hbm.at[g, pl.ds(band * G * NB, G * NB)])

    plsc.subcore_barrier()

    @pl.loop(0, ITEMS_PER_SUB)
    def _phase2(it):
        item = sub * ITEMS_PER_SUB + it
        il = item // G
        band = item % G
        g = core * IMG_PER_CORE + il
        lo = jnp.clip(band - 1, 0, G - 3)
        bandlow = jnp.clip(band - 1, 0, G - 2)
        pltpu.sync_copy(lut_hbm.at[g, pl.ds(lo * G * NB, 3 * G * NB)], luti_v)
        pltpu.sync_copy(img_hbm.at[g, pl.ds(band * K, K), :], pix_v)
        for j in (0, 1):
            r0 = bandlow + j - lo
            r1 = jnp.minimum(bandlow + j + 1, G - 1) - lo
            for ckk in range(G * NB // LANE):
                av = luti_v[pl.ds(r0 * (G * NB) + ckk * LANE, LANE)]
                bv = luti_v[pl.ds(r1 * (G * NB) + ckk * LANE, LANE)]
                p4_v[pl.ds(j * (G * NB) + ckk * LANE, LANE)] = jnp.bitwise_or(
                    av, lax.shift_left(bv, 16))

        wx_zero = jnp.zeros((LANE,), jnp.float32)
        wx_pat = [wx_v[pl.ds((2 + j) * LANE, LANE)] for j in range(4)]

        @plsc.parallel_loop(0, K, 1, unroll=3)
        def _rows(r):
            y = band * K + r
            ty = jnp.clip((y.astype(jnp.float32) + 0.5) * (1.0 / K) - 0.5,
                          0.0, G - 1.0)
            t0 = jnp.clip(lax.shift_right_arithmetic(2 * y - (K - 1), 7),
                          0, G - 1)
            wy = ty - t0.astype(jnp.float32)
            base = (t0 - bandlow) * (G * NB)
            for ck in range(CHUNKS):
                x0 = ck * LANE
                s0c = 0 if x0 < K // 2 else min((x0 - K // 2) // K, G - 1)
                px = pix_v[r, pl.ds(ck * LANE, LANE)]
                pb = (px * NB).astype(jnp.int32)
                wx = (wx_zero if ck < 2 or ck >= CHUNKS - 2
                      else wx_pat[(ck - 2) % 4])
                g4 = plsc.load_gather(
                    p4_v.at[pl.ds(base + s0c * NB, NB)], [pb])
                v00 = jnp.bitwise_and(g4, 255).astype(jnp.float32)
                v01 = jnp.bitwise_and(
                    lax.shift_right_logical(g4, 8), 255).astype(jnp.float32)
                v10 = jnp.bitwise_and(
                    lax.shift_right_logical(g4, 16), 255).astype(jnp.float32)
                v11 = lax.shift_right_logical(g4, 24).astype(jnp.float32)
                top = v00 + wx * (v01 - v00)
                bot = v10 + wx * (v11 - v10)
                outb_v[r, pl.ds(ck * LANE, LANE)] = (
                    (top + wy * (bot - top)) * (1.0 / (NB - 1.0)))

        pltpu.sync_copy(outb_v, out_hbm.at[g, pl.ds(band * K, K), :])


@jax.jit
def _clahe_sc(img3):
    fn = pl.kernel(
        _body,
        out_type=(jax.ShapeDtypeStruct((NIMG, H, W), jnp.float32),
                  jax.ShapeDtypeStruct((NIMG, LUT_WORDS), jnp.int32)),
        mesh=plsc.VectorSubcoreMesh(core_axis_name="c", subcore_axis_name="s"),
        compiler_params=pltpu.CompilerParams(needs_layout_passes=False),
        scratch_types=[
            pltpu.VMEM((K, W), jnp.float32),
            pltpu.VMEM((K, W), jnp.float32),
            pltpu.VMEM((G * NB,), jnp.float32),
            pltpu.VMEM((G * NB,), jnp.int32),
            pltpu.VMEM((G * NB,), jnp.int32),
            pltpu.VMEM((3 * G * NB,), jnp.int32),
            pltpu.VMEM((2 * G * NB,), jnp.int32),
            pltpu.VMEM((W,), jnp.float32),
        ],
    )
    return fn(img3)[0]


def kernel(img):
    out = _clahe_sc(img.reshape(NIMG, H, W))
    return out.reshape(B, C, H, W)


# speedup vs baseline: 1.0091x; 1.0091x over previous
import functools

import jax
import jax.numpy as jnp
from jax import lax
from jax.experimental import pallas as pl
from jax.experimental.pallas import tpu as pltpu
from jax.experimental.pallas import tpu_sc as plsc

B, C, H, W = 8, 3, 512, 512
G = 8
K = 64
NB = 256
PIX = K * K
MAXV = 640.0
SCALE = (NB - 1.0) / PIX
NIMG = B * C
NCORE, NSUB = 2, 16
IMG_PER_CORE = NIMG // NCORE
ITEMS_PER_SUB = IMG_PER_CORE * G // NSUB
LANE = 16
CHUNKS = W // LANE
LUT_WORDS = G * G * NB


def _body(img_hbm, out_hbm, lut_hbm, pix_v, outb_v, hist_v, lutb_v, ph_v,
          luti_v, p4_v, wx_v):
    core = lax.axis_index("c")
    sub = lax.axis_index("s")

    for ck in range(CHUNKS):
        xx = lax.iota(jnp.int32, LANE).astype(jnp.float32) + float(ck * LANE)
        sx = jnp.clip((xx + 0.5) * (1.0 / K) - 0.5, 0.0, G - 1.0)
        s0 = sx.astype(jnp.int32)
        wx_v[pl.ds(ck * LANE, LANE)] = sx - s0.astype(jnp.float32)

    zeros = jnp.zeros((LANE,), jnp.float32)
    ones = jnp.ones((LANE,), jnp.float32)

    @pl.loop(0, ITEMS_PER_SUB)
    def _phase1(it):
        item = sub * ITEMS_PER_SUB + it
        il = item // G
        band = item % G
        g = core * IMG_PER_CORE + il
        pltpu.sync_copy(img_hbm.at[g, pl.ds(band * K, K), :], pix_v)
        for i in range(G * NB // LANE):
            hist_v[pl.ds(i * LANE, LANE)] = zeros

        @plsc.parallel_loop(0, K, 1, unroll=2)
        def _rows(r):
            for ckx in range(CHUNKS):
                ck = (ckx % 8) * 4 + ckx // 8
                px = pix_v[r, pl.ds(ck * LANE, LANE)]
                pb = (px * NB).astype(jnp.int32)
                plsc.addupdate_scatter(
                    hist_v.at[pl.ds((ck // 4) * NB, NB)], [pb], ones)

        for t in range(G):
            acc = zeros
            for i in range(NB // LANE):
                h = jnp.minimum(hist_v[pl.ds(t * NB + i * LANE, LANE)], MAXV)
                acc = acc + h
            tot = jnp.sum(acc)
            clipped_i = PIX - tot.astype(jnp.int32)
            redist_i = lax.shift_right_logical(clipped_i, 8)
            redist = redist_i.astype(jnp.float32)
            residual = (clipped_i - redist_i * NB).astype(jnp.float32)
            carry = 0.0
            for i in range(NB // LANE):
                h = jnp.minimum(hist_v[pl.ds(t * NB + i * LANE, LANE)], MAXV)
                vr = lax.iota(jnp.int32, LANE).astype(jnp.float32) + float(i * LANE)
                h = h + redist + jnp.where(vr < residual, 1.0, 0.0)
                cs = plsc.cumsum(h) + carry
                carry = carry + jnp.sum(h)
                lut = jnp.clip(cs * SCALE, 0.0, NB - 1.0)
                lutb_v[pl.ds(t * NB + i * LANE, LANE)] = lut.astype(jnp.int32)
        for ckk in range(G * NB // LANE):
            tx = ckk // (NB // LANE)
            off = (ckk % (NB // LANE)) * LANE
            av = lutb_v[pl.ds(tx * NB + off, LANE)]
            bv = lutb_v[pl.ds(min(tx + 1, G - 1) * NB + off, LANE)]
            ph_v[pl.ds(ckk * LANE, LANE)] = jnp.bitwise_or(
                av, lax.shift_left(bv, 8))
        pltpu.sync_copy(ph_v, lut_hbm.at[g, pl.ds(band * G * NB, G * NB)])

    plsc.subcore_barrier()

    @pl.loop(0, ITEMS_PER_SUB)
    def _phase2(it):
        item = sub * ITEMS_PER_SUB + it
        il = item // G
        band = item % G
        g = core * IMG_PER_CORE + il
        lo = jnp.clip(band - 1, 0, G - 3)
        bandlow = jnp.clip(band - 1, 0, G - 2)
        pltpu.sync_copy(lut_hbm.at[g, pl.ds(lo * G * NB, 3 * G * NB)], luti_v)
        pltpu.sync_copy(img_hbm.at[g, pl.ds(band * K, K), :], pix_v)
        for j in (0, 1):
            r0 = bandlow + j - lo
            r1 = jnp.minimum(bandlow + j + 1, G - 1) - lo
            for ckk in range(G * NB // LANE):
                av = luti_v[pl.ds(r0 * (G * NB) + ckk * LANE, LANE)]
                bv = luti_v[pl.ds(r1 * (G * NB) + ckk * LANE, LANE)]
                p4_v[pl.ds(j * (G * NB) + ckk * LANE, LANE)] = jnp.bitwise_or(
                    av, lax.shift_left(bv, 16))

        wx_zero = jnp.zeros((LANE,), jnp.float32)
        wx_pat = [wx_v[pl.ds((2 + j) * LANE, LANE)] for j in range(4)]

        @plsc.parallel_loop(0, K, 1, unroll=3)
        def _rows(r):
            y = band * K + r
            ty = jnp.clip((y.astype(jnp.float32) + 0.5) * (1.0 / K) - 0.5,
                          0.0, G - 1.0)
            t0 = jnp.clip(lax.shift_right_arithmetic(2 * y - (K - 1), 7),
                          0, G - 1)
            wy = ty - t0.astype(jnp.float32)
            base = (t0 - bandlow) * (G * NB)
            for ck in range(CHUNKS):
                x0 = ck * LANE
                s0c = 0 if x0 < K // 2 else min((x0 - K // 2) // K, G - 1)
                px = pix_v[r, pl.ds(ck * LANE, LANE)]
                pb = (px * NB).astype(jnp.int32)
                wx = (wx_zero if ck < 2 or ck >= CHUNKS - 2
                      else wx_pat[(ck - 2) % 4])
                g4 = plsc.load_gather(
                    p4_v.at[pl.ds(base + s0c * NB, NB)], [pb])
                v00 = jnp.bitwise_and(g4, 255).astype(jnp.float32)
                v01 = jnp.bitwise_and(
                    lax.shift_right_logical(g4, 8), 255).astype(jnp.float32)
                v10 = jnp.bitwise_and(
                    lax.shift_right_logical(g4, 16), 255).astype(jnp.float32)
                v11 = lax.shift_right_logical(g4, 24).astype(jnp.float32)
                top = v00 + wx * (v01 - v00)
                bot = v10 + wx * (v11 - v10)
                outb_v[r, pl.ds(ck * LANE, LANE)] = (
                    (top + wy * (bot - top)) * (1.0 / (NB - 1.0)))

        pltpu.sync_copy(outb_v, out_hbm.at[g, pl.ds(band * K, K), :])


@jax.jit
def _clahe_sc(img3):
    fn = pl.kernel(
        _body,
        out_type=(jax.ShapeDtypeStruct((NIMG, H, W), jnp.float32),
                  jax.ShapeDtypeStruct((NIMG, LUT_WORDS), jnp.int32)),
        mesh=plsc.VectorSubcoreMesh(core_axis_name="c", subcore_axis_name="s"),
        compiler_params=pltpu.CompilerParams(needs_layout_passes=False),
        scratch_types=[
            pltpu.VMEM((K, W), jnp.float32),
            pltpu.VMEM((K, W), jnp.float32),
            pltpu.VMEM((G * NB,), jnp.float32),
            pltpu.VMEM((G * NB,), jnp.int32),
            pltpu.VMEM((G * NB,), jnp.int32),
            pltpu.VMEM((3 * G * NB,), jnp.int32),
            pltpu.VMEM((2 * G * NB,), jnp.int32),
            pltpu.VMEM((W,), jnp.float32),
        ],
    )
    return fn(img3)[0]


def kernel(img):
    out = _clahe_sc(img.reshape(NIMG, H, W))
    return out.reshape(B, C, H, W)
